# baseline (speedup 1.0000x reference)
import functools
import math

import jax
import jax.numpy as jnp
from jax import lax
from jax.experimental import pallas as pl
from jax.experimental.pallas import tpu as pltpu
from jax.experimental.pallas import tpu_sc as plsc

_BN = 1.0 / math.sqrt(1.0 + 1e-5)

_L = 16
_NW = 32
_BIG = 1 << 30


def _sc_mesh():
    return plsc.VectorSubcoreMesh(core_axis_name="c", subcore_axis_name="s")


def _iota16():
    return lax.iota(jnp.int32, _L)


def _at(ref, i):
    return ref[pl.ds(i, _L)][0]


def _put(ref, i, val):
    plsc.store_scatter(ref, [jnp.full((_L,), i, jnp.int32)],
                       jnp.full((_L,), val), mask=_iota16() == 0)


def _bf16r(v):
    u = lax.bitcast_convert_type(v, jnp.int32)
    r = (u + 0x7FFF + ((u >> 16) & 1)) & (-65536)
    return lax.bitcast_convert_type(r, jnp.float32)


def _fps_sc(src_x, src_y, src_z, dist, out_x, out_y, out_z, n_src, n_samp):
    nch = n_src // _L

    def init(j, c):
        dist[pl.ds(j * _L, _L)] = jnp.full((_L,), 1e10, jnp.float32)
        return c
    lax.fori_loop(0, nch, init, 0)

    def step(s, far):
        cx = _at(src_x, far)
        cy = _at(src_y, far)
        cz = _at(src_z, far)
        _put(out_x, s, cx)
        _put(out_y, s, cy)
        _put(out_z, s, cz)

        def chunk(j, carry):
            vmax, vidx = carry
            sl = pl.ds(j * _L, _L)
            dx = src_x[sl] - cx
            dy = src_y[sl] - cy
            dz = src_z[sl] - cz
            d = (dx * dx + dy * dy) + dz * dz
            dnew = jnp.minimum(dist[sl], d)
            dist[sl] = dnew
            upd = dnew > vmax
            vmax = jnp.where(upd, dnew, vmax)
            vidx = jnp.where(upd, _iota16() + j * _L, vidx)
            return vmax, vidx

        vmax, vidx = lax.fori_loop(
            0, nch, chunk,
            (jnp.full((_L,), -1.0, jnp.float32), jnp.zeros((_L,), jnp.int32)),
            unroll=8)
        gmax = jnp.max(vmax)
        cand = jnp.where(vmax == gmax, vidx, _BIG)
        return jnp.min(cand)

    lax.fori_loop(0, n_samp, step, jnp.int32(0))


def _bq_sc(r2, n_samp, n_cent, rb_x, rb_y, rb_z, src_sq,
           n_src, cen_x, cen_y, cen_z, sel, emit):
    nch = n_src // _L

    def center(c, _):
        cx = _at(cen_x, c)
        cy = _at(cen_y, c)
        cz = _at(cen_z, c)
        asq = (cx * cx + cy * cy) + cz * cz
        cxb = _bf16r(cx)
        cyb = _bf16r(cy)
        czb = _bf16r(cz)

        def chunk(j, cnt):
            sl = pl.ds(j * _L, _L)
            xv = rb_x[sl]
            yv = rb_y[sl]
            zv = rb_z[sl]
            dot = (cxb * xv + cyb * yv) + czb * zv
            d = (-2.0 * dot + asq) + src_sq[sl]
            m = d <= r2
            incl = plsc.cumsum(jnp.where(m, 1, 0))
            m2 = m & (cnt + incl <= n_samp)
            plsc.store_scatter(sel, [cnt + (incl - 1)],
                               _iota16() + j * _L, mask=m2)
            tot = jnp.max(incl)
            return jnp.minimum(cnt + tot, n_samp)

        U = 4

        def group(carry):
            g, cnt = carry
            for u in range(U):
                cnt = chunk(g * U + u, cnt)
            return g + 1, cnt

        _, cnt = lax.while_loop(
            lambda carry: (carry[0] < nch // U) & (carry[1] < n_samp),
            group, (jnp.int32(0), jnp.int32(0)))
        first = sel[pl.ds(0, _L)][0]
        for k in range(n_samp // _L):
            pos = _iota16() + k * _L
            sv = sel[pl.ds(k * _L, _L)]
            sv = jnp.where(pos < cnt, sv, first)
            emit(c, k, sv, cx, cy, cz)
        return 0

    lax.fori_loop(0, n_cent, center, 0)


def _geo_body(xs_h, ys_h, zs_h,
              d1x_h, d1y_h, d1z_h, i2_h, d2x_h, d2y_h, d2z_h,
              nx2_h, ny2_h, nz2_h,
              xs, ys, zs, psq, dist, nx1, ny1, nz1, nsq, sel,
              rbx, rby, rbz,
              d1x, d1y, d1z, i2, d2x, d2y, d2z, nx2, ny2, nz2):
    w = lax.axis_index("s") * 2 + lax.axis_index("c")
    pltpu.sync_copy(xs_h.at[w], xs.at[pl.ds(0, 1024)])
    pltpu.sync_copy(ys_h.at[w], ys.at[pl.ds(0, 1024)])
    pltpu.sync_copy(zs_h.at[w], zs.at[pl.ds(0, 1024)])

    def pchunk(j, c):
        sl = pl.ds(j * _L, _L)
        xv = xs[sl]
        yv = ys[sl]
        zv = zs[sl]
        psq[sl] = (xv * xv + yv * yv) + zv * zv
        return c
    lax.fori_loop(0, 64, pchunk, 0)

    _fps_sc(xs, ys, zs, dist, nx1, ny1, nz1, 1024, 512)

    def r1chunk(j, c):
        sl = pl.ds(j * _L, _L)
        rbx[sl] = _bf16r(xs[sl])
        rby[sl] = _bf16r(ys[sl])
        rbz[sl] = _bf16r(zs[sl])
        return c
    lax.fori_loop(0, 64, r1chunk, 0)

    def emit1(c, k, sv, cx, cy, cz):
        gx = plsc.load_gather(xs, [sv])
        gy = plsc.load_gather(ys, [sv])
        gz = plsc.load_gather(zs, [sv])
        o = pl.ds(c * 32 + k * _L, _L)
        d1x[o] = gx - cx
        d1y[o] = gy - cy
        d1z[o] = gz - cz

    _bq_sc(jnp.float32(0.2 ** 2), 32, 512, rbx, rby, rbz, psq,
           1024, nx1, ny1, nz1, sel, emit1)

    def nchunk(j, c):
        sl = pl.ds(j * _L, _L)
        xv = nx1[sl]
        yv = ny1[sl]
        zv = nz1[sl]
        nsq[sl] = (xv * xv + yv * yv) + zv * zv
        return c
    lax.fori_loop(0, 32, nchunk, 0)

    _fps_sc(nx1, ny1, nz1, dist, nx2, ny2, nz2, 512, 128)

    def r2chunk(j, c):
        sl = pl.ds(j * _L, _L)
        rbx[sl] = _bf16r(nx1[sl])
        rby[sl] = _bf16r(ny1[sl])
        rbz[sl] = _bf16r(nz1[sl])
        return c
    lax.fori_loop(0, 32, r2chunk, 0)

    base = w * 512

    def emit2(c, k, sv, cx, cy, cz):
        gx = plsc.load_gather(nx1, [sv])
        gy = plsc.load_gather(ny1, [sv])
        gz = plsc.load_gather(nz1, [sv])
        o = pl.ds(c * 64 + k * _L, _L)
        d2x[o] = gx - cx
        d2y[o] = gy - cy
        d2z[o] = gz - cz
        i2[o] = sv + base

    _bq_sc(jnp.float32(0.4 ** 2), 64, 128, rbx, rby, rbz, nsq,
           512, nx2, ny2, nz2, sel, emit2)

    pltpu.sync_copy(d1x, d1x_h.at[w])
    pltpu.sync_copy(d1y, d1y_h.at[w])
    pltpu.sync_copy(d1z, d1z_h.at[w])
    pltpu.sync_copy(i2, i2_h.at[w])
    pltpu.sync_copy(d2x, d2x_h.at[w])
    pltpu.sync_copy(d2y, d2y_h.at[w])
    pltpu.sync_copy(d2z, d2z_h.at[w])
    pltpu.sync_copy(nx2.at[pl.ds(0, 128)], nx2_h.at[w])
    pltpu.sync_copy(ny2.at[pl.ds(0, 128)], ny2_h.at[w])
    pltpu.sync_copy(nz2.at[pl.ds(0, 128)], nz2_h.at[w])


def _geometry(xs, ys, zs):
    f32 = jnp.float32
    outs = [
        jax.ShapeDtypeStruct((32, 16384), f32),
        jax.ShapeDtypeStruct((32, 16384), f32),
        jax.ShapeDtypeStruct((32, 16384), f32),
        jax.ShapeDtypeStruct((32, 8192), jnp.int32),
        jax.ShapeDtypeStruct((32, 8192), f32),
        jax.ShapeDtypeStruct((32, 8192), f32),
        jax.ShapeDtypeStruct((32, 8192), f32),
        jax.ShapeDtypeStruct((32, 128), f32),
        jax.ShapeDtypeStruct((32, 128), f32),
        jax.ShapeDtypeStruct((32, 128), f32),
    ]
    scratch = [
        pltpu.VMEM((1040,), f32), pltpu.VMEM((1040,), f32),
        pltpu.VMEM((1040,), f32), pltpu.VMEM((1024,), f32),
        pltpu.VMEM((1024,), f32),
        pltpu.VMEM((528,), f32), pltpu.VMEM((528,), f32),
        pltpu.VMEM((528,), f32), pltpu.VMEM((512,), f32),
        pltpu.VMEM((80,), jnp.int32),
        pltpu.VMEM((1024,), f32), pltpu.VMEM((1024,), f32),
        pltpu.VMEM((1024,), f32),
        pltpu.VMEM((16384,), f32), pltpu.VMEM((16384,), f32),
        pltpu.VMEM((16384,), f32),
        pltpu.VMEM((8192,), jnp.int32),
        pltpu.VMEM((8192,), f32), pltpu.VMEM((8192,), f32),
        pltpu.VMEM((8192,), f32),
        pltpu.VMEM((144,), f32), pltpu.VMEM((144,), f32),
        pltpu.VMEM((144,), f32),
    ]
    fn = pl.kernel(_geo_body, mesh=_sc_mesh(), out_type=outs,
                   scratch_types=scratch,
                   compiler_params=pltpu.CompilerParams(
                       needs_layout_passes=False))
    return fn(xs, ys, zs)


def _gather_body(tab_h, idx_h, out_h, idx_v, rows_v, sem):
    w = lax.axis_index("s") * 2 + lax.axis_index("c")

    def it(i, c):
        b = w * 8192 + i * 512
        pltpu.sync_copy(idx_h.at[pl.ds(b, 512)], idx_v)
        pltpu.async_copy(tab_h.at[idx_v], rows_v, sem).wait()
        pltpu.sync_copy(rows_v, out_h.at[pl.ds(b, 512)])
        return c
    lax.fori_loop(0, 16, it, 0)


def _gather_rows(table, idx):
    fn = pl.kernel(
        _gather_body, mesh=_sc_mesh(),
        out_type=jax.ShapeDtypeStruct((262144, 128), jnp.float32),
        scratch_types=[
            pltpu.VMEM((512,), jnp.int32),
            pltpu.VMEM((512, 128), jnp.float32),
            pltpu.SemaphoreType.DMA,
        ])
    return fn(table, idx)


def _mlp1_body(x_ref, w0, b0, w1, b1, w2, b2, o_ref):
    x = x_ref[0]
    h = jax.nn.relu((jnp.dot(x, w0[...], preferred_element_type=jnp.float32)
                     + b0[...]) * _BN)
    h = jax.nn.relu((jnp.dot(h, w1[...], preferred_element_type=jnp.float32)
                     + b1[...]) * _BN)
    h = jax.nn.relu((jnp.dot(h, w2[...], preferred_element_type=jnp.float32)
                     + b2[...]) * _BN)
    bm = h.shape[0]
    o_ref[0] = jnp.max(h.reshape(bm // 32, 32, 128), axis=1)


def _mlp1(grouped, w0, b0, w1, b1, w2, b2):
    BM = 4096
    grid = (32, 16384 // BM)
    return pl.pallas_call(
        _mlp1_body,
        grid=grid,
        in_specs=[
            pl.BlockSpec((1, BM, 3), lambda b, m: (b, m, 0)),
            pl.BlockSpec((3, 64), lambda b, m: (0, 0)),
            pl.BlockSpec((64,), lambda b, m: (0,)),
            pl.BlockSpec((64, 64), lambda b, m: (0, 0)),
            pl.BlockSpec((64,), lambda b, m: (0,)),
            pl.BlockSpec((64, 128), lambda b, m: (0, 0)),
            pl.BlockSpec((128,), lambda b, m: (0,)),
        ],
        out_specs=pl.BlockSpec((1, BM // 32, 128), lambda b, m: (b, m, 0)),
        out_shape=jax.ShapeDtypeStruct((32, 512, 128), jnp.float32),
        compiler_params=pltpu.CompilerParams(
            dimension_semantics=("parallel", "parallel")),
    )(grouped, w0, b0, w1, b1, w2, b2)


def _mlp2_body(d_ref, g_ref, w0x, w0f, b0, w1, b1, w2, b2, o_ref):
    d = d_ref[0]
    g = g_ref[0]
    h = (jnp.dot(d, w0x[...], preferred_element_type=jnp.float32)
         + jnp.dot(g, w0f[...], preferred_element_type=jnp.float32))
    h = jax.nn.relu((h + b0[...]) * _BN)
    h = jax.nn.relu((jnp.dot(h, w1[...], preferred_element_type=jnp.float32)
                     + b1[...]) * _BN)
    h = jax.nn.relu((jnp.dot(h, w2[...], preferred_element_type=jnp.float32)
                     + b2[...]) * _BN)
    bm = h.shape[0]
    o_ref[0] = jnp.max(h.reshape(bm // 64, 64, 256), axis=1)


def _mlp2(diffs, gfeat, w0x, w0f, b0, w1, b1, w2, b2):
    BM = 4096
    grid = (32, 8192 // BM)
    return pl.pallas_call(
        _mlp2_body,
        grid=grid,
        in_specs=[
            pl.BlockSpec((1, BM, 3), lambda b, m: (b, m, 0)),
            pl.BlockSpec((1, BM, 128), lambda b, m: (b, m, 0)),
            pl.BlockSpec((3, 128), lambda b, m: (0, 0)),
            pl.BlockSpec((128, 128), lambda b, m: (0, 0)),
            pl.BlockSpec((128,), lambda b, m: (0,)),
            pl.BlockSpec((128, 128), lambda b, m: (0, 0)),
            pl.BlockSpec((128,), lambda b, m: (0,)),
            pl.BlockSpec((128, 256), lambda b, m: (0, 0)),
            pl.BlockSpec((256,), lambda b, m: (0,)),
        ],
        out_specs=pl.BlockSpec((1, BM // 64, 256), lambda b, m: (b, m, 0)),
        out_shape=jax.ShapeDtypeStruct((32, 128, 256), jnp.float32),
        compiler_params=pltpu.CompilerParams(
            dimension_semantics=("parallel", "parallel")),
    )(diffs, gfeat, w0x, w0f, b0, w1, b1, w2, b2)


def _head_body(xyz_ref, p_ref, w0x, w0f, b0, w1, b1, w2, b2,
               f1w, f1b, f2w, f2b, f3w, f3b, pred_ref, feat_ref):
    x = xyz_ref[0]
    p = p_ref[0]
    h = (jnp.dot(x, w0x[...], preferred_element_type=jnp.float32)
         + jnp.dot(p, w0f[...], preferred_element_type=jnp.float32))
    h = jax.nn.relu((h + b0[...]) * _BN)
    h = jax.nn.relu((jnp.dot(h, w1[...], preferred_element_type=jnp.float32)
                     + b1[...]) * _BN)
    h = jax.nn.relu((jnp.dot(h, w2[...], preferred_element_type=jnp.float32)
                     + b2[...]) * _BN)
    feat = jnp.max(h, axis=0, keepdims=True)
    feat_ref[0] = feat
    z = jax.nn.relu((jnp.dot(feat, f1w[...], preferred_element_type=jnp.float32)
                     + f1b[...]) * _BN)
    z = jax.nn.relu((jnp.dot(z, f2w[...], preferred_element_type=jnp.float32)
                     + f2b[...]) * _BN)
    z = jnp.dot(z, f3w[...], preferred_element_type=jnp.float32) + f3b[...]
    z = z - jnp.max(z, axis=-1, keepdims=True)
    z = z - jnp.log(jnp.sum(jnp.exp(z), axis=-1, keepdims=True))
    pred_ref[0] = z


def _head(l2_xyz, l2_points, w0x, w0f, b0, w1, b1, w2, b2,
          f1w, f1b, f2w, f2b, f3w, f3b):
    grid = (32,)
    return pl.pallas_call(
        _head_body,
        grid=grid,
        in_specs=[
            pl.BlockSpec((1, 128, 3), lambda b: (b, 0, 0)),
            pl.BlockSpec((1, 128, 256), lambda b: (b, 0, 0)),
            pl.BlockSpec((3, 256), lambda b: (0, 0)),
            pl.BlockSpec((256, 256), lambda b: (0, 0)),
            pl.BlockSpec((256,), lambda b: (0,)),
            pl.BlockSpec((256, 512), lambda b: (0, 0)),
            pl.BlockSpec((512,), lambda b: (0,)),
            pl.BlockSpec((512, 1024), lambda b: (0, 0)),
            pl.BlockSpec((1024,), lambda b: (0,)),
            pl.BlockSpec((1024, 512), lambda b: (0, 0)),
            pl.BlockSpec((512,), lambda b: (0,)),
            pl.BlockSpec((512, 256), lambda b: (0, 0)),
            pl.BlockSpec((256,), lambda b: (0,)),
            pl.BlockSpec((256, 40), lambda b: (0, 0)),
            pl.BlockSpec((40,), lambda b: (0,)),
        ],
        out_specs=[
            pl.BlockSpec((1, 1, 40), lambda b: (b, 0, 0)),
            pl.BlockSpec((1, 1, 1024), lambda b: (b, 0, 0)),
        ],
        out_shape=[
            jax.ShapeDtypeStruct((32, 1, 40), jnp.float32),
            jax.ShapeDtypeStruct((32, 1, 1024), jnp.float32),
        ],
        compiler_params=pltpu.CompilerParams(
            dimension_semantics=("parallel",)),
    )(l2_xyz, l2_points, w0x, w0f, b0, w1, b1, w2, b2,
      f1w, f1b, f2w, f2b, f3w, f3b)


def kernel(xyz, sa1_w0, sa1_b0, sa1_w1, sa1_b1, sa1_w2, sa1_b2,
           sa2_w0, sa2_b0, sa2_w1, sa2_b1, sa2_w2, sa2_b2,
           sa3_w0, sa3_b0, sa3_w1, sa3_b1, sa3_w2, sa3_b2,
           fc1_w, fc1_b, fc2_w, fc2_b, fc3_w, fc3_b):
    B, t, d, n = xyz.shape
    BT = B * t
    xs = xyz.reshape(BT, d, n)

    (d1x, d1y, d1z, i2, d2x, d2y, d2z, nx2, ny2, nz2) = _geometry(
        xs[:, 0], xs[:, 1], xs[:, 2])

    g1 = jnp.stack([d1x, d1y, d1z], axis=-1)

    l1 = _mlp1(g1, sa1_w0, sa1_b0, sa1_w1, sa1_b1, sa1_w2, sa1_b2)

    new2 = jnp.stack([nx2, ny2, nz2], axis=-1)
    d2 = jnp.stack([d2x, d2y, d2z], axis=-1)
    g2 = _gather_rows(l1.reshape(BT * 512, 128),
                      i2.reshape(BT * 8192)).reshape(BT, 8192, 128)

    l2 = _mlp2(d2, g2, sa2_w0[:3], sa2_w0[3:], sa2_b0,
               sa2_w1, sa2_b1, sa2_w2, sa2_b2)

    pred, feat = _head(new2, l2, sa3_w0[:3], sa3_w0[3:], sa3_b0,
                       sa3_w1, sa3_b1, sa3_w2, sa3_b2,
                       fc1_w, fc1_b, fc2_w, fc2_b, fc3_w, fc3_b)

    pred = jnp.transpose(pred.reshape(B, t, 40), (0, 2, 1))
    feat = feat.reshape(B, t, 1024)
    return pred, feat


# speedup vs baseline: 1.0447x; 1.0447x over previous
import functools
import math

import jax
import jax.numpy as jnp
from jax import lax
from jax.experimental import pallas as pl
from jax.experimental.pallas import tpu as pltpu
from jax.experimental.pallas import tpu_sc as plsc

_BN = 1.0 / math.sqrt(1.0 + 1e-5)

_L = 16
_NW = 32
_BIG = 1 << 30


def _sc_mesh():
    return plsc.VectorSubcoreMesh(core_axis_name="c", subcore_axis_name="s")


def _iota16():
    return lax.iota(jnp.int32, _L)


def _at(ref, i):
    return ref[pl.ds(i, _L)][0]


def _put(ref, i, val):
    plsc.store_scatter(ref, [jnp.full((_L,), i, jnp.int32)],
                       jnp.full((_L,), val), mask=_iota16() == 0)


def _bf16r(v):
    u = lax.bitcast_convert_type(v, jnp.int32)
    r = (u + 0x7FFF + ((u >> 16) & 1)) & (-65536)
    return lax.bitcast_convert_type(r, jnp.float32)


def _fps_sc(src_x, src_y, src_z, dist, out_x, out_y, out_z, n_src, n_samp):
    nch = n_src // _L

    def init(j, c):
        dist[pl.ds(j * _L, _L)] = jnp.full((_L,), 1e10, jnp.float32)
        return c
    lax.fori_loop(0, nch, init, 0)

    def step(s, far):
        cx = _at(src_x, far)
        cy = _at(src_y, far)
        cz = _at(src_z, far)
        _put(out_x, s, cx)
        _put(out_y, s, cy)
        _put(out_z, s, cz)

        def chunk(j, carry):
            vmax, vidx = carry
            sl = pl.ds(j * _L, _L)
            dx = src_x[sl] - cx
            dy = src_y[sl] - cy
            dz = src_z[sl] - cz
            d = (dx * dx + dy * dy) + dz * dz
            dnew = jnp.minimum(dist[sl], d)
            dist[sl] = dnew
            upd = dnew > vmax
            vmax = jnp.where(upd, dnew, vmax)
            vidx = jnp.where(upd, _iota16() + j * _L, vidx)
            return vmax, vidx

        vmax, vidx = lax.fori_loop(
            0, nch, chunk,
            (jnp.full((_L,), -1.0, jnp.float32), jnp.zeros((_L,), jnp.int32)),
            unroll=8)
        gmax = jnp.max(vmax)
        cand = jnp.where(vmax == gmax, vidx, _BIG)
        return jnp.min(cand)

    lax.fori_loop(0, n_samp, step, jnp.int32(0))


def _bq_sc(r2, n_samp, n_cent, rb_x, rb_y, rb_z, src_sq,
           n_src, cen_x, cen_y, cen_z, sel, emit):
    nch = n_src // _L

    def center(c, _):
        cx = _at(cen_x, c)
        cy = _at(cen_y, c)
        cz = _at(cen_z, c)
        asq = (cx * cx + cy * cy) + cz * cz
        cxb = _bf16r(cx)
        cyb = _bf16r(cy)
        czb = _bf16r(cz)

        def chunk(j, cnt):
            sl = pl.ds(j * _L, _L)
            xv = rb_x[sl]
            yv = rb_y[sl]
            zv = rb_z[sl]
            dot = (cxb * xv + cyb * yv) + czb * zv
            d = (-2.0 * dot + asq) + src_sq[sl]
            m = d <= r2
            incl = plsc.cumsum(jnp.where(m, 1, 0))
            m2 = m & (cnt + incl <= n_samp)
            plsc.store_scatter(sel, [cnt + (incl - 1)],
                               _iota16() + j * _L, mask=m2)
            tot = jnp.max(incl)
            return jnp.minimum(cnt + tot, n_samp)

        cnt = lax.fori_loop(0, nch, chunk, jnp.int32(0), unroll=4)
        first = sel[pl.ds(0, _L)][0]
        for k in range(n_samp // _L):
            pos = _iota16() + k * _L
            sv = sel[pl.ds(k * _L, _L)]
            sv = jnp.where(pos < cnt, sv, first)
            emit(c, k, sv, cx, cy, cz)
        return 0

    lax.fori_loop(0, n_cent, center, 0)


def _geo_body(xs_h, ys_h, zs_h,
              d1x_h, d1y_h, d1z_h, i2_h, d2x_h, d2y_h, d2z_h,
              nx2_h, ny2_h, nz2_h,
              xs, ys, zs, psq, dist, nx1, ny1, nz1, nsq, sel,
              rbx, rby, rbz,
              d1x, d1y, d1z, i2, d2x, d2y, d2z, nx2, ny2, nz2):
    w = lax.axis_index("s") * 2 + lax.axis_index("c")
    pltpu.sync_copy(xs_h.at[w], xs.at[pl.ds(0, 1024)])
    pltpu.sync_copy(ys_h.at[w], ys.at[pl.ds(0, 1024)])
    pltpu.sync_copy(zs_h.at[w], zs.at[pl.ds(0, 1024)])

    def pchunk(j, c):
        sl = pl.ds(j * _L, _L)
        xv = xs[sl]
        yv = ys[sl]
        zv = zs[sl]
        psq[sl] = (xv * xv + yv * yv) + zv * zv
        return c
    lax.fori_loop(0, 64, pchunk, 0)

    _fps_sc(xs, ys, zs, dist, nx1, ny1, nz1, 1024, 512)

    def r1chunk(j, c):
        sl = pl.ds(j * _L, _L)
        rbx[sl] = _bf16r(xs[sl])
        rby[sl] = _bf16r(ys[sl])
        rbz[sl] = _bf16r(zs[sl])
        return c
    lax.fori_loop(0, 64, r1chunk, 0)

    def emit1(c, k, sv, cx, cy, cz):
        gx = plsc.load_gather(xs, [sv])
        gy = plsc.load_gather(ys, [sv])
        gz = plsc.load_gather(zs, [sv])
        o = pl.ds(c * 32 + k * _L, _L)
        d1x[o] = gx - cx
        d1y[o] = gy - cy
        d1z[o] = gz - cz

    _bq_sc(jnp.float32(0.2 ** 2), 32, 512, rbx, rby, rbz, psq,
           1024, nx1, ny1, nz1, sel, emit1)

    def nchunk(j, c):
        sl = pl.ds(j * _L, _L)
        xv = nx1[sl]
        yv = ny1[sl]
        zv = nz1[sl]
        nsq[sl] = (xv * xv + yv * yv) + zv * zv
        return c
    lax.fori_loop(0, 32, nchunk, 0)

    _fps_sc(nx1, ny1, nz1, dist, nx2, ny2, nz2, 512, 128)

    def r2chunk(j, c):
        sl = pl.ds(j * _L, _L)
        rbx[sl] = _bf16r(nx1[sl])
        rby[sl] = _bf16r(ny1[sl])
        rbz[sl] = _bf16r(nz1[sl])
        return c
    lax.fori_loop(0, 32, r2chunk, 0)

    base = w * 512

    def emit2(c, k, sv, cx, cy, cz):
        gx = plsc.load_gather(nx1, [sv])
        gy = plsc.load_gather(ny1, [sv])
        gz = plsc.load_gather(nz1, [sv])
        o = pl.ds(c * 64 + k * _L, _L)
        d2x[o] = gx - cx
        d2y[o] = gy - cy
        d2z[o] = gz - cz
        i2[o] = sv + base

    _bq_sc(jnp.float32(0.4 ** 2), 64, 128, rbx, rby, rbz, nsq,
           512, nx2, ny2, nz2, sel, emit2)

    pltpu.sync_copy(d1x, d1x_h.at[w])
    pltpu.sync_copy(d1y, d1y_h.at[w])
    pltpu.sync_copy(d1z, d1z_h.at[w])
    pltpu.sync_copy(i2, i2_h.at[w])
    pltpu.sync_copy(d2x, d2x_h.at[w])
    pltpu.sync_copy(d2y, d2y_h.at[w])
    pltpu.sync_copy(d2z, d2z_h.at[w])
    pltpu.sync_copy(nx2.at[pl.ds(0, 128)], nx2_h.at[w])
    pltpu.sync_copy(ny2.at[pl.ds(0, 128)], ny2_h.at[w])
    pltpu.sync_copy(nz2.at[pl.ds(0, 128)], nz2_h.at[w])


def _geometry(xs, ys, zs):
    f32 = jnp.float32
    outs = [
        jax.ShapeDtypeStruct((32, 16384), f32),
        jax.ShapeDtypeStruct((32, 16384), f32),
        jax.ShapeDtypeStruct((32, 16384), f32),
        jax.ShapeDtypeStruct((32, 8192), jnp.int32),
        jax.ShapeDtypeStruct((32, 8192), f32),
        jax.ShapeDtypeStruct((32, 8192), f32),
        jax.ShapeDtypeStruct((32, 8192), f32),
        jax.ShapeDtypeStruct((32, 128), f32),
        jax.ShapeDtypeStruct((32, 128), f32),
        jax.ShapeDtypeStruct((32, 128), f32),
    ]
    scratch = [
        pltpu.VMEM((1040,), f32), pltpu.VMEM((1040,), f32),
        pltpu.VMEM((1040,), f32), pltpu.VMEM((1024,), f32),
        pltpu.VMEM((1024,), f32),
        pltpu.VMEM((528,), f32), pltpu.VMEM((528,), f32),
        pltpu.VMEM((528,), f32), pltpu.VMEM((512,), f32),
        pltpu.VMEM((80,), jnp.int32),
        pltpu.VMEM((1024,), f32), pltpu.VMEM((1024,), f32),
        pltpu.VMEM((1024,), f32),
        pltpu.VMEM((16384,), f32), pltpu.VMEM((16384,), f32),
        pltpu.VMEM((16384,), f32),
        pltpu.VMEM((8192,), jnp.int32),
        pltpu.VMEM((8192,), f32), pltpu.VMEM((8192,), f32),
        pltpu.VMEM((8192,), f32),
        pltpu.VMEM((144,), f32), pltpu.VMEM((144,), f32),
        pltpu.VMEM((144,), f32),
    ]
    fn = pl.kernel(_geo_body, mesh=_sc_mesh(), out_type=outs,
                   scratch_types=scratch,
                   compiler_params=pltpu.CompilerParams(
                       needs_layout_passes=False))
    return fn(xs, ys, zs)


def _gather_body(tab_h, idx_h, out_h, idx_v, rows_v, sem):
    w = lax.axis_index("s") * 2 + lax.axis_index("c")

    def it(i, c):
        b = w * 8192 + i * 512
        pltpu.sync_copy(idx_h.at[pl.ds(b, 512)], idx_v)
        pltpu.async_copy(tab_h.at[idx_v], rows_v, sem).wait()
        pltpu.sync_copy(rows_v, out_h.at[pl.ds(b, 512)])
        return c
    lax.fori_loop(0, 16, it, 0)


def _gather_rows(table, idx):
    fn = pl.kernel(
        _gather_body, mesh=_sc_mesh(),
        out_type=jax.ShapeDtypeStruct((262144, 128), jnp.float32),
        scratch_types=[
            pltpu.VMEM((512,), jnp.int32),
            pltpu.VMEM((512, 128), jnp.float32),
            pltpu.SemaphoreType.DMA,
        ])
    return fn(table, idx)


def _mlp1_body(x_ref, w0, b0, w1, b1, w2, b2, o_ref):
    x = x_ref[0]
    h = jax.nn.relu((jnp.dot(x, w0[...], preferred_element_type=jnp.float32)
                     + b0[...]) * _BN)
    h = jax.nn.relu((jnp.dot(h, w1[...], preferred_element_type=jnp.float32)
                     + b1[...]) * _BN)
    h = jax.nn.relu((jnp.dot(h, w2[...], preferred_element_type=jnp.float32)
                     + b2[...]) * _BN)
    bm = h.shape[0]
    o_ref[0] = jnp.max(h.reshape(bm // 32, 32, 128), axis=1)


def _mlp1(grouped, w0, b0, w1, b1, w2, b2):
    BM = 4096
    grid = (32, 16384 // BM)
    return pl.pallas_call(
        _mlp1_body,
        grid=grid,
        in_specs=[
            pl.BlockSpec((1, BM, 3), lambda b, m: (b, m, 0)),
            pl.BlockSpec((3, 64), lambda b, m: (0, 0)),
            pl.BlockSpec((64,), lambda b, m: (0,)),
            pl.BlockSpec((64, 64), lambda b, m: (0, 0)),
            pl.BlockSpec((64,), lambda b, m: (0,)),
            pl.BlockSpec((64, 128), lambda b, m: (0, 0)),
            pl.BlockSpec((128,), lambda b, m: (0,)),
        ],
        out_specs=pl.BlockSpec((1, BM // 32, 128), lambda b, m: (b, m, 0)),
        out_shape=jax.ShapeDtypeStruct((32, 512, 128), jnp.float32),
        compiler_params=pltpu.CompilerParams(
            dimension_semantics=("parallel", "parallel")),
    )(grouped, w0, b0, w1, b1, w2, b2)


def _mlp2_body(d_ref, g_ref, w0x, w0f, b0, w1, b1, w2, b2, o_ref):
    d = d_ref[0]
    g = g_ref[0]
    h = (jnp.dot(d, w0x[...], preferred_element_type=jnp.float32)
         + jnp.dot(g, w0f[...], preferred_element_type=jnp.float32))
    h = jax.nn.relu((h + b0[...]) * _BN)
    h = jax.nn.relu((jnp.dot(h, w1[...], preferred_element_type=jnp.float32)
                     + b1[...]) * _BN)
    h = jax.nn.relu((jnp.dot(h, w2[...], preferred_element_type=jnp.float32)
                     + b2[...]) * _BN)
    bm = h.shape[0]
    o_ref[0] = jnp.max(h.reshape(bm // 64, 64, 256), axis=1)


def _mlp2(diffs, gfeat, w0x, w0f, b0, w1, b1, w2, b2):
    BM = 4096
    grid = (32, 8192 // BM)
    return pl.pallas_call(
        _mlp2_body,
        grid=grid,
        in_specs=[
            pl.BlockSpec((1, BM, 3), lambda b, m: (b, m, 0)),
            pl.BlockSpec((1, BM, 128), lambda b, m: (b, m, 0)),
            pl.BlockSpec((3, 128), lambda b, m: (0, 0)),
            pl.BlockSpec((128, 128), lambda b, m: (0, 0)),
            pl.BlockSpec((128,), lambda b, m: (0,)),
            pl.BlockSpec((128, 128), lambda b, m: (0, 0)),
            pl.BlockSpec((128,), lambda b, m: (0,)),
            pl.BlockSpec((128, 256), lambda b, m: (0, 0)),
            pl.BlockSpec((256,), lambda b, m: (0,)),
        ],
        out_specs=pl.BlockSpec((1, BM // 64, 256), lambda b, m: (b, m, 0)),
        out_shape=jax.ShapeDtypeStruct((32, 128, 256), jnp.float32),
        compiler_params=pltpu.CompilerParams(
            dimension_semantics=("parallel", "parallel")),
    )(diffs, gfeat, w0x, w0f, b0, w1, b1, w2, b2)


def _head_body(xyz_ref, p_ref, w0x, w0f, b0, w1, b1, w2, b2,
               f1w, f1b, f2w, f2b, f3w, f3b, pred_ref, feat_ref):
    x = xyz_ref[0]
    p = p_ref[0]
    h = (jnp.dot(x, w0x[...], preferred_element_type=jnp.float32)
         + jnp.dot(p, w0f[...], preferred_element_type=jnp.float32))
    h = jax.nn.relu((h + b0[...]) * _BN)
    h = jax.nn.relu((jnp.dot(h, w1[...], preferred_element_type=jnp.float32)
                     + b1[...]) * _BN)
    h = jax.nn.relu((jnp.dot(h, w2[...], preferred_element_type=jnp.float32)
                     + b2[...]) * _BN)
    feat = jnp.max(h, axis=0, keepdims=True)
    feat_ref[0] = feat
    z = jax.nn.relu((jnp.dot(feat, f1w[...], preferred_element_type=jnp.float32)
                     + f1b[...]) * _BN)
    z = jax.nn.relu((jnp.dot(z, f2w[...], preferred_element_type=jnp.float32)
                     + f2b[...]) * _BN)
    z = jnp.dot(z, f3w[...], preferred_element_type=jnp.float32) + f3b[...]
    z = z - jnp.max(z, axis=-1, keepdims=True)
    z = z - jnp.log(jnp.sum(jnp.exp(z), axis=-1, keepdims=True))
    pred_ref[0] = z


def _head(l2_xyz, l2_points, w0x, w0f, b0, w1, b1, w2, b2,
          f1w, f1b, f2w, f2b, f3w, f3b):
    grid = (32,)
    return pl.pallas_call(
        _head_body,
        grid=grid,
        in_specs=[
            pl.BlockSpec((1, 128, 3), lambda b: (b, 0, 0)),
            pl.BlockSpec((1, 128, 256), lambda b: (b, 0, 0)),
            pl.BlockSpec((3, 256), lambda b: (0, 0)),
            pl.BlockSpec((256, 256), lambda b: (0, 0)),
            pl.BlockSpec((256,), lambda b: (0,)),
            pl.BlockSpec((256, 512), lambda b: (0, 0)),
            pl.BlockSpec((512,), lambda b: (0,)),
            pl.BlockSpec((512, 1024), lambda b: (0, 0)),
            pl.BlockSpec((1024,), lambda b: (0,)),
            pl.BlockSpec((1024, 512), lambda b: (0, 0)),
            pl.BlockSpec((512,), lambda b: (0,)),
            pl.BlockSpec((512, 256), lambda b: (0, 0)),
            pl.BlockSpec((256,), lambda b: (0,)),
            pl.BlockSpec((256, 40), lambda b: (0, 0)),
            pl.BlockSpec((40,), lambda b: (0,)),
        ],
        out_specs=[
            pl.BlockSpec((1, 1, 40), lambda b: (b, 0, 0)),
            pl.BlockSpec((1, 1, 1024), lambda b: (b, 0, 0)),
        ],
        out_shape=[
            jax.ShapeDtypeStruct((32, 1, 40), jnp.float32),
            jax.ShapeDtypeStruct((32, 1, 1024), jnp.float32),
        ],
        compiler_params=pltpu.CompilerParams(
            dimension_semantics=("parallel",)),
    )(l2_xyz, l2_points, w0x, w0f, b0, w1, b1, w2, b2,
      f1w, f1b, f2w, f2b, f3w, f3b)


def kernel(xyz, sa1_w0, sa1_b0, sa1_w1, sa1_b1, sa1_w2, sa1_b2,
           sa2_w0, sa2_b0, sa2_w1, sa2_b1, sa2_w2, sa2_b2,
           sa3_w0, sa3_b0, sa3_w1, sa3_b1, sa3_w2, sa3_b2,
           fc1_w, fc1_b, fc2_w, fc2_b, fc3_w, fc3_b):
    B, t, d, n = xyz.shape
    BT = B * t
    xs = xyz.reshape(BT, d, n)

    (d1x, d1y, d1z, i2, d2x, d2y, d2z, nx2, ny2, nz2) = _geometry(
        xs[:, 0], xs[:, 1], xs[:, 2])

    g1 = jnp.stack([d1x, d1y, d1z], axis=-1)

    l1 = _mlp1(g1, sa1_w0, sa1_b0, sa1_w1, sa1_b1, sa1_w2, sa1_b2)

    new2 = jnp.stack([nx2, ny2, nz2], axis=-1)
    d2 = jnp.stack([d2x, d2y, d2z], axis=-1)
    g2 = _gather_rows(l1.reshape(BT * 512, 128),
                      i2.reshape(BT * 8192)).reshape(BT, 8192, 128)

    l2 = _mlp2(d2, g2, sa2_w0[:3], sa2_w0[3:], sa2_b0,
               sa2_w1, sa2_b1, sa2_w2, sa2_b2)

    pred, feat = _head(new2, l2, sa3_w0[:3], sa3_w0[3:], sa3_b0,
                       sa3_w1, sa3_b1, sa3_w2, sa3_b2,
                       fc1_w, fc1_b, fc2_w, fc2_b, fc3_w, fc3_b)

    pred = jnp.transpose(pred.reshape(B, t, 40), (0, 2, 1))
    feat = feat.reshape(B, t, 1024)
    return pred, feat


# speedup vs baseline: 1.0546x; 1.0094x over previous
import functools
import math

import jax
import jax.numpy as jnp
from jax import lax
from jax.experimental import pallas as pl
from jax.experimental.pallas import tpu as pltpu
from jax.experimental.pallas import tpu_sc as plsc

_BN = 1.0 / math.sqrt(1.0 + 1e-5)

_L = 16
_NW = 32
_BIG = 1 << 30


def _sc_mesh():
    return plsc.VectorSubcoreMesh(core_axis_name="c", subcore_axis_name="s")


def _iota16():
    return lax.iota(jnp.int32, _L)


def _at(ref, i):
    return ref[pl.ds(i, _L)][0]


def _put(ref, i, val):
    plsc.store_scatter(ref, [jnp.full((_L,), i, jnp.int32)],
                       jnp.full((_L,), val), mask=_iota16() == 0)


def _bf16r(v):
    u = lax.bitcast_convert_type(v, jnp.int32)
    r = (u + 0x7FFF + ((u >> 16) & 1)) & (-65536)
    return lax.bitcast_convert_type(r, jnp.float32)


def _fps_sc(src_x, src_y, src_z, dist, out_x, out_y, out_z, n_src, n_samp):
    nch = n_src // _L

    def init(j, c):
        dist[pl.ds(j * _L, _L)] = jnp.full((_L,), 1e10, jnp.float32)
        return c
    lax.fori_loop(0, nch, init, 0)

    def step(s, far):
        cx = _at(src_x, far)
        cy = _at(src_y, far)
        cz = _at(src_z, far)
        _put(out_x, s, cx)
        _put(out_y, s, cy)
        _put(out_z, s, cz)

        def chunk(j, carry):
            vmax, vidx = carry
            sl = pl.ds(j * _L, _L)
            dx = src_x[sl] - cx
            dy = src_y[sl] - cy
            dz = src_z[sl] - cz
            d = (dx * dx + dy * dy) + dz * dz
            dnew = jnp.minimum(dist[sl], d)
            dist[sl] = dnew
            upd = dnew > vmax
            vmax = jnp.where(upd, dnew, vmax)
            vidx = jnp.where(upd, _iota16() + j * _L, vidx)
            return vmax, vidx

        vmax, vidx = lax.fori_loop(
            0, nch, chunk,
            (jnp.full((_L,), -1.0, jnp.float32), jnp.zeros((_L,), jnp.int32)),
            unroll=8)
        gmax = jnp.max(vmax)
        cand = jnp.where(vmax == gmax, vidx, _BIG)
        return jnp.min(cand)

    lax.fori_loop(0, n_samp, step, jnp.int32(0))


def _bq_sc(r2, n_samp, n_cent, rb_x, rb_y, rb_z, src_sq,
           n_src, cen_x, cen_y, cen_z, sel, emit):
    nch = n_src // _L

    def center(c, _):
        cx = _at(cen_x, c)
        cy = _at(cen_y, c)
        cz = _at(cen_z, c)
        asq = (cx * cx + cy * cy) + cz * cz
        cxb = _bf16r(cx)
        cyb = _bf16r(cy)
        czb = _bf16r(cz)

        def chunk(j, cnt):
            sl = pl.ds(j * _L, _L)
            xv = rb_x[sl]
            yv = rb_y[sl]
            zv = rb_z[sl]
            dot = (cxb * xv + cyb * yv) + czb * zv
            d = (-2.0 * dot + asq) + src_sq[sl]
            m = d <= r2
            incl = plsc.cumsum(jnp.where(m, 1, 0))
            m2 = m & (cnt + incl <= n_samp)
            plsc.store_scatter(sel, [cnt + (incl - 1)],
                               _iota16() + j * _L, mask=m2)
            tot = jnp.max(incl)
            return jnp.minimum(cnt + tot, n_samp)

        cnt = lax.fori_loop(0, nch, chunk, jnp.int32(0), unroll=4)
        first = sel[pl.ds(0, _L)][0]
        for k in range(n_samp // _L):
            pos = _iota16() + k * _L
            sv = sel[pl.ds(k * _L, _L)]
            sv = jnp.where(pos < cnt, sv, first)
            emit(c, k, sv, cx, cy, cz)
        return 0

    lax.fori_loop(0, n_cent, center, 0)


def _geo_body(xs_h, ys_h, zs_h,
              d1x_h, d1y_h, d1z_h, i2_h, d2x_h, d2y_h, d2z_h,
              nx2_h, ny2_h, nz2_h,
              xs, ys, zs, psq, dist, nx1, ny1, nz1, nsq, sel,
              rbx, rby, rbz,
              d1x, d1y, d1z, i2, d2x, d2y, d2z, nx2, ny2, nz2):
    w = lax.axis_index("s") * 2 + lax.axis_index("c")
    pltpu.sync_copy(xs_h.at[w], xs.at[pl.ds(0, 1024)])
    pltpu.sync_copy(ys_h.at[w], ys.at[pl.ds(0, 1024)])
    pltpu.sync_copy(zs_h.at[w], zs.at[pl.ds(0, 1024)])

    def pchunk(j, c):
        sl = pl.ds(j * _L, _L)
        xv = xs[sl]
        yv = ys[sl]
        zv = zs[sl]
        psq[sl] = (xv * xv + yv * yv) + zv * zv
        return c
    lax.fori_loop(0, 64, pchunk, 0)

    _fps_sc(xs, ys, zs, dist, nx1, ny1, nz1, 1024, 512)

    def r1chunk(j, c):
        sl = pl.ds(j * _L, _L)
        rbx[sl] = _bf16r(xs[sl])
        rby[sl] = _bf16r(ys[sl])
        rbz[sl] = _bf16r(zs[sl])
        return c
    lax.fori_loop(0, 64, r1chunk, 0)

    def emit1(c, k, sv, cx, cy, cz):
        gx = plsc.load_gather(xs, [sv])
        gy = plsc.load_gather(ys, [sv])
        gz = plsc.load_gather(zs, [sv])
        o = pl.ds(c * 32 + k * _L, _L)
        d1x[o] = gx - cx
        d1y[o] = gy - cy
        d1z[o] = gz - cz

    _bq_sc(jnp.float32(0.2 ** 2), 32, 512, rbx, rby, rbz, psq,
           1024, nx1, ny1, nz1, sel, emit1)

    def nchunk(j, c):
        sl = pl.ds(j * _L, _L)
        xv = nx1[sl]
        yv = ny1[sl]
        zv = nz1[sl]
        nsq[sl] = (xv * xv + yv * yv) + zv * zv
        return c
    lax.fori_loop(0, 32, nchunk, 0)

    _fps_sc(nx1, ny1, nz1, dist, nx2, ny2, nz2, 512, 128)

    def r2chunk(j, c):
        sl = pl.ds(j * _L, _L)
        rbx[sl] = _bf16r(nx1[sl])
        rby[sl] = _bf16r(ny1[sl])
        rbz[sl] = _bf16r(nz1[sl])
        return c
    lax.fori_loop(0, 32, r2chunk, 0)

    base = w * 512

    def emit2(c, k, sv, cx, cy, cz):
        gx = plsc.load_gather(nx1, [sv])
        gy = plsc.load_gather(ny1, [sv])
        gz = plsc.load_gather(nz1, [sv])
        o = pl.ds(c * 64 + k * _L, _L)
        d2x[o] = gx - cx
        d2y[o] = gy - cy
        d2z[o] = gz - cz
        i2[o] = sv + base

    _bq_sc(jnp.float32(0.4 ** 2), 64, 128, rbx, rby, rbz, nsq,
           512, nx2, ny2, nz2, sel, emit2)

    pltpu.sync_copy(d1x, d1x_h.at[w])
    pltpu.sync_copy(d1y, d1y_h.at[w])
    pltpu.sync_copy(d1z, d1z_h.at[w])
    pltpu.sync_copy(i2, i2_h.at[w])
    pltpu.sync_copy(d2x, d2x_h.at[w])
    pltpu.sync_copy(d2y, d2y_h.at[w])
    pltpu.sync_copy(d2z, d2z_h.at[w])
    pltpu.sync_copy(nx2.at[pl.ds(0, 128)], nx2_h.at[w])
    pltpu.sync_copy(ny2.at[pl.ds(0, 128)], ny2_h.at[w])
    pltpu.sync_copy(nz2.at[pl.ds(0, 128)], nz2_h.at[w])


def _geometry(xs, ys, zs):
    f32 = jnp.float32
    outs = [
        jax.ShapeDtypeStruct((32, 16384), f32),
        jax.ShapeDtypeStruct((32, 16384), f32),
        jax.ShapeDtypeStruct((32, 16384), f32),
        jax.ShapeDtypeStruct((32, 8192), jnp.int32),
        jax.ShapeDtypeStruct((32, 8192), f32),
        jax.ShapeDtypeStruct((32, 8192), f32),
        jax.ShapeDtypeStruct((32, 8192), f32),
        jax.ShapeDtypeStruct((32, 128), f32),
        jax.ShapeDtypeStruct((32, 128), f32),
        jax.ShapeDtypeStruct((32, 128), f32),
    ]
    scratch = [
        pltpu.VMEM((1040,), f32), pltpu.VMEM((1040,), f32),
        pltpu.VMEM((1040,), f32), pltpu.VMEM((1024,), f32),
        pltpu.VMEM((1024,), f32),
        pltpu.VMEM((528,), f32), pltpu.VMEM((528,), f32),
        pltpu.VMEM((528,), f32), pltpu.VMEM((512,), f32),
        pltpu.VMEM((80,), jnp.int32),
        pltpu.VMEM((1024,), f32), pltpu.VMEM((1024,), f32),
        pltpu.VMEM((1024,), f32),
        pltpu.VMEM((16384,), f32), pltpu.VMEM((16384,), f32),
        pltpu.VMEM((16384,), f32),
        pltpu.VMEM((8192,), jnp.int32),
        pltpu.VMEM((8192,), f32), pltpu.VMEM((8192,), f32),
        pltpu.VMEM((8192,), f32),
        pltpu.VMEM((144,), f32), pltpu.VMEM((144,), f32),
        pltpu.VMEM((144,), f32),
    ]
    fn = pl.kernel(_geo_body, mesh=_sc_mesh(), out_type=outs,
                   scratch_types=scratch,
                   compiler_params=pltpu.CompilerParams(
                       needs_layout_passes=False))
    return fn(xs, ys, zs)


def _gather_body(tab_h, idx_h, out_h, iv0, iv1, rv0, rv1, gs0, gs1, ss0, ss1):
    w = lax.axis_index("s") * 2 + lax.axis_index("c")
    NCH = 32
    CS = 256
    idx_v = [iv0, iv1]
    rows_v = [rv0, rv1]
    gsem = [gs0, gs1]
    ssem = [ss0, ss1]
    gh = [None, None]
    sh = [None, None]
    for g in range(NCH + 1):
        b = g % 2
        if g < NCH:
            if sh[b] is not None:
                sh[b].wait()
            off = w * 8192 + g * CS
            pltpu.sync_copy(idx_h.at[pl.ds(off, CS)], idx_v[b])
            gh[b] = pltpu.async_copy(tab_h.at[idx_v[b]], rows_v[b], gsem[b])
        if g >= 1:
            pb = (g - 1) % 2
            gh[pb].wait()
            poff = w * 8192 + (g - 1) * CS
            sh[pb] = pltpu.async_copy(rows_v[pb],
                                      out_h.at[pl.ds(poff, CS)], ssem[pb])
    sh[0].wait()
    sh[1].wait()


def _gather_rows(table, idx):
    fn = pl.kernel(
        _gather_body, mesh=_sc_mesh(),
        out_type=jax.ShapeDtypeStruct((262144, 128), jnp.float32),
        scratch_types=[
            pltpu.VMEM((256,), jnp.int32), pltpu.VMEM((256,), jnp.int32),
            pltpu.VMEM((256, 128), jnp.float32),
            pltpu.VMEM((256, 128), jnp.float32),
            pltpu.SemaphoreType.DMA, pltpu.SemaphoreType.DMA,
            pltpu.SemaphoreType.DMA, pltpu.SemaphoreType.DMA,
        ])
    return fn(table, idx)


def _mlp1_body(x_ref, w0, b0, w1, b1, w2, b2, o_ref):
    x = x_ref[0]
    h = jax.nn.relu((jnp.dot(x, w0[...], preferred_element_type=jnp.float32)
                     + b0[...]) * _BN)
    h = jax.nn.relu((jnp.dot(h, w1[...], preferred_element_type=jnp.float32)
                     + b1[...]) * _BN)
    h = jax.nn.relu((jnp.dot(h, w2[...], preferred_element_type=jnp.float32)
                     + b2[...]) * _BN)
    bm = h.shape[0]
    o_ref[0] = jnp.max(h.reshape(bm // 32, 32, 128), axis=1)


def _mlp1(grouped, w0, b0, w1, b1, w2, b2):
    BM = 4096
    grid = (32, 16384 // BM)
    return pl.pallas_call(
        _mlp1_body,
        grid=grid,
        in_specs=[
            pl.BlockSpec((1, BM, 3), lambda b, m: (b, m, 0)),
            pl.BlockSpec((3, 64), lambda b, m: (0, 0)),
            pl.BlockSpec((64,), lambda b, m: (0,)),
            pl.BlockSpec((64, 64), lambda b, m: (0, 0)),
            pl.BlockSpec((64,), lambda b, m: (0,)),
            pl.BlockSpec((64, 128), lambda b, m: (0, 0)),
            pl.BlockSpec((128,), lambda b, m: (0,)),
        ],
        out_specs=pl.BlockSpec((1, BM // 32, 128), lambda b, m: (b, m, 0)),
        out_shape=jax.ShapeDtypeStruct((32, 512, 128), jnp.float32),
        compiler_params=pltpu.CompilerParams(
            dimension_semantics=("parallel", "parallel")),
    )(grouped, w0, b0, w1, b1, w2, b2)


def _mlp2_body(d_ref, g_ref, w0x, w0f, b0, w1, b1, w2, b2, o_ref):
    d = d_ref[0]
    g = g_ref[0]
    h = (jnp.dot(d, w0x[...], preferred_element_type=jnp.float32)
         + jnp.dot(g, w0f[...], preferred_element_type=jnp.float32))
    h = jax.nn.relu((h + b0[...]) * _BN)
    h = jax.nn.relu((jnp.dot(h, w1[...], preferred_element_type=jnp.float32)
                     + b1[...]) * _BN)
    h = jax.nn.relu((jnp.dot(h, w2[...], preferred_element_type=jnp.float32)
                     + b2[...]) * _BN)
    bm = h.shape[0]
    o_ref[0] = jnp.max(h.reshape(bm // 64, 64, 256), axis=1)


def _mlp2(diffs, gfeat, w0x, w0f, b0, w1, b1, w2, b2):
    BM = 4096
    grid = (32, 8192 // BM)
    return pl.pallas_call(
        _mlp2_body,
        grid=grid,
        in_specs=[
            pl.BlockSpec((1, BM, 3), lambda b, m: (b, m, 0)),
            pl.BlockSpec((1, BM, 128), lambda b, m: (b, m, 0)),
            pl.BlockSpec((3, 128), lambda b, m: (0, 0)),
            pl.BlockSpec((128, 128), lambda b, m: (0, 0)),
            pl.BlockSpec((128,), lambda b, m: (0,)),
            pl.BlockSpec((128, 128), lambda b, m: (0, 0)),
            pl.BlockSpec((128,), lambda b, m: (0,)),
            pl.BlockSpec((128, 256), lambda b, m: (0, 0)),
            pl.BlockSpec((256,), lambda b, m: (0,)),
        ],
        out_specs=pl.BlockSpec((1, BM // 64, 256), lambda b, m: (b, m, 0)),
        out_shape=jax.ShapeDtypeStruct((32, 128, 256), jnp.float32),
        compiler_params=pltpu.CompilerParams(
            dimension_semantics=("parallel", "parallel")),
    )(diffs, gfeat, w0x, w0f, b0, w1, b1, w2, b2)


def _head_body(xyz_ref, p_ref, w0x, w0f, b0, w1, b1, w2, b2,
               f1w, f1b, f2w, f2b, f3w, f3b, pred_ref, feat_ref):
    x = xyz_ref[0]
    p = p_ref[0]
    h = (jnp.dot(x, w0x[...], preferred_element_type=jnp.float32)
         + jnp.dot(p, w0f[...], preferred_element_type=jnp.float32))
    h = jax.nn.relu((h + b0[...]) * _BN)
    h = jax.nn.relu((jnp.dot(h, w1[...], preferred_element_type=jnp.float32)
                     + b1[...]) * _BN)
    h = jax.nn.relu((jnp.dot(h, w2[...], preferred_element_type=jnp.float32)
                     + b2[...]) * _BN)
    feat = jnp.max(h, axis=0, keepdims=True)
    feat_ref[0] = feat
    z = jax.nn.relu((jnp.dot(feat, f1w[...], preferred_element_type=jnp.float32)
                     + f1b[...]) * _BN)
    z = jax.nn.relu((jnp.dot(z, f2w[...], preferred_element_type=jnp.float32)
                     + f2b[...]) * _BN)
    z = jnp.dot(z, f3w[...], preferred_element_type=jnp.float32) + f3b[...]
    z = z - jnp.max(z, axis=-1, keepdims=True)
    z = z - jnp.log(jnp.sum(jnp.exp(z), axis=-1, keepdims=True))
    pred_ref[0] = z


def _head(l2_xyz, l2_points, w0x, w0f, b0, w1, b1, w2, b2,
          f1w, f1b, f2w, f2b, f3w, f3b):
    grid = (32,)
    return pl.pallas_call(
        _head_body,
        grid=grid,
        in_specs=[
            pl.BlockSpec((1, 128, 3), lambda b: (b, 0, 0)),
            pl.BlockSpec((1, 128, 256), lambda b: (b, 0, 0)),
            pl.BlockSpec((3, 256), lambda b: (0, 0)),
            pl.BlockSpec((256, 256), lambda b: (0, 0)),
            pl.BlockSpec((256,), lambda b: (0,)),
            pl.BlockSpec((256, 512), lambda b: (0, 0)),
            pl.BlockSpec((512,), lambda b: (0,)),
            pl.BlockSpec((512, 1024), lambda b: (0, 0)),
            pl.BlockSpec((1024,), lambda b: (0,)),
            pl.BlockSpec((1024, 512), lambda b: (0, 0)),
            pl.BlockSpec((512,), lambda b: (0,)),
            pl.BlockSpec((512, 256), lambda b: (0, 0)),
            pl.BlockSpec((256,), lambda b: (0,)),
            pl.BlockSpec((256, 40), lambda b: (0, 0)),
            pl.BlockSpec((40,), lambda b: (0,)),
        ],
        out_specs=[
            pl.BlockSpec((1, 1, 40), lambda b: (b, 0, 0)),
            pl.BlockSpec((1, 1, 1024), lambda b: (b, 0, 0)),
        ],
        out_shape=[
            jax.ShapeDtypeStruct((32, 1, 40), jnp.float32),
            jax.ShapeDtypeStruct((32, 1, 1024), jnp.float32),
        ],
        compiler_params=pltpu.CompilerParams(
            dimension_semantics=("parallel",)),
    )(l2_xyz, l2_points, w0x, w0f, b0, w1, b1, w2, b2,
      f1w, f1b, f2w, f2b, f3w, f3b)


def kernel(xyz, sa1_w0, sa1_b0, sa1_w1, sa1_b1, sa1_w2, sa1_b2,
           sa2_w0, sa2_b0, sa2_w1, sa2_b1, sa2_w2, sa2_b2,
           sa3_w0, sa3_b0, sa3_w1, sa3_b1, sa3_w2, sa3_b2,
           fc1_w, fc1_b, fc2_w, fc2_b, fc3_w, fc3_b):
    B, t, d, n = xyz.shape
    BT = B * t
    xs = xyz.reshape(BT, d, n)

    (d1x, d1y, d1z, i2, d2x, d2y, d2z, nx2, ny2, nz2) = _geometry(
        xs[:, 0], xs[:, 1], xs[:, 2])

    g1 = jnp.stack([d1x, d1y, d1z], axis=-1)

    l1 = _mlp1(g1, sa1_w0, sa1_b0, sa1_w1, sa1_b1, sa1_w2, sa1_b2)

    new2 = jnp.stack([nx2, ny2, nz2], axis=-1)
    d2 = jnp.stack([d2x, d2y, d2z], axis=-1)
    g2 = _gather_rows(l1.reshape(BT * 512, 128),
                      i2.reshape(BT * 8192)).reshape(BT, 8192, 128)

    l2 = _mlp2(d2, g2, sa2_w0[:3], sa2_w0[3:], sa2_b0,
               sa2_w1, sa2_b1, sa2_w2, sa2_b2)

    pred, feat = _head(new2, l2, sa3_w0[:3], sa3_w0[3:], sa3_b0,
                       sa3_w1, sa3_b1, sa3_w2, sa3_b2,
                       fc1_w, fc1_b, fc2_w, fc2_b, fc3_w, fc3_b)

    pred = jnp.transpose(pred.reshape(B, t, 40), (0, 2, 1))
    feat = feat.reshape(B, t, 1024)
    return pred, feat
